# Initial kernel scaffold; baseline (speedup 1.0000x reference)
import functools

import jax
import jax.numpy as jnp
from jax import lax
from jax.experimental import pallas as pl
from jax.experimental.pallas import tpu as pltpu
from jax.experimental.pallas import tpu_sc as plsc

_N = 10000
_E = 320000
_D = 128
_PD = 64
_DEPTH = 16
_B = 1024
_K = 32
_TEMP = 0.5
_LAMBDA1 = 1e-05

_NPAD = 10016
_EPT = 10240
_EPAD = 32 * _EPT
_CH = 128
_NCHUNK = _EPT // _CH
_ZROWS = _NPAD // 16
_G = _B * (_K + 2)
_GPT = _G // 32
_GCH = 64
_GNCH = _GPT // _GCH

_BN = 2000
_BB = 256


def _mm_nt(a, b):
    return lax.dot_general(a, b, (((1,), (1,)), ((), ())),
                           preferred_element_type=jnp.float32)


def _mm_nn(a, b):
    return lax.dot_general(a, b, (((1,), (0,)), ((), ())),
                           preferred_element_type=jnp.float32)


def _proj0_body(es_ref, oh_ref, epw_ref, pw_ref, pb_ref, w0_ref, b0_ref,
                x0_ref, h0_ref):
    pw = pw_ref[...]
    wa = pw[:, :_D]
    wb = pw[:, _D:]
    emb_pp = _mm_nt(epw_ref[...], wb)
    x0 = _mm_nt(es_ref[...], wa) + _mm_nn(oh_ref[...], emb_pp) + pb_ref[...]
    h0 = jnp.maximum(_mm_nt(x0, w0_ref[...]) + b0_ref[...], 0.0)
    x0_ref[...] = x0
    h0_ref[...] = h0


def _proj0(emb_s, onehot, emb_p_w, proj_W, pb2, W0, b02):
    grid = (_N // _BN,)
    return pl.pallas_call(
        _proj0_body,
        grid=grid,
        in_specs=[
            pl.BlockSpec((_BN, _D), lambda i: (i, 0)),
            pl.BlockSpec((_BN, _DEPTH), lambda i: (i, 0)),
            pl.BlockSpec((_DEPTH, _PD), lambda i: (0, 0)),
            pl.BlockSpec((_D, _D + _PD), lambda i: (0, 0)),
            pl.BlockSpec((1, _D), lambda i: (0, 0)),
            pl.BlockSpec((_D, _D), lambda i: (0, 0)),
            pl.BlockSpec((1, _D), lambda i: (0, 0)),
        ],
        out_specs=[
            pl.BlockSpec((_BN, _D), lambda i: (i, 0)),
            pl.BlockSpec((_BN, _D), lambda i: (i, 0)),
        ],
        out_shape=[
            jax.ShapeDtypeStruct((_N, _D), jnp.float32),
            jax.ShapeDtypeStruct((_N, _D), jnp.float32),
        ],
    )(emb_s, onehot, emb_p_w, proj_W, pb2, W0, b02)


def _res_body_both(x_ref, p0_ref, p1_ref, w_ref, b_ref, xn_ref, y_ref):
    xn = x_ref[...] + p0_ref[...] + p1_ref[...]
    y = jnp.maximum(_mm_nt(xn, w_ref[...]) + b_ref[...], 0.0)
    xn_ref[...] = xn
    y_ref[...] = y


def _res_body_out(x_ref, p0_ref, p1_ref, w_ref, b_ref, y_ref):
    xn = x_ref[...] + p0_ref[...] + p1_ref[...]
    y_ref[...] = _mm_nt(xn, w_ref[...]) + b_ref[...]


def _residual_layer(x, p0, p1, W, b2, want_xn):
    grid = (_N // _BN,)
    in_specs = [
        pl.BlockSpec((_BN, _D), lambda i: (i, 0)),
        pl.BlockSpec((_BN, _D), lambda i: (i, 0)),
        pl.BlockSpec((_BN, _D), lambda i: (i, 0)),
        pl.BlockSpec((_D, _D), lambda i: (0, 0)),
        pl.BlockSpec((1, _D), lambda i: (0, 0)),
    ]
    if want_xn:
        return pl.pallas_call(
            _res_body_both,
            grid=grid,
            in_specs=in_specs,
            out_specs=[pl.BlockSpec((_BN, _D), lambda i: (i, 0))] * 2,
            out_shape=[jax.ShapeDtypeStruct((_N, _D), jnp.float32)] * 2,
        )(x, p0, p1, W, b2)
    return pl.pallas_call(
        _res_body_out,
        grid=grid,
        in_specs=in_specs,
        out_specs=pl.BlockSpec((_BN, _D), lambda i: (i, 0)),
        out_shape=jax.ShapeDtypeStruct((_N, _D), jnp.float32),
    )(x, p0, p1, W, b2)


_SC_MESH = plsc.VectorSubcoreMesh(core_axis_name="c", subcore_axis_name="s")


@functools.partial(
    pl.kernel,
    out_type=jax.ShapeDtypeStruct((2, _NPAD, _D), jnp.float32),
    mesh=_SC_MESH,
    scratch_types=[
        pltpu.VMEM_SHARED((_NPAD, _D), jnp.float32),
        pltpu.VMEM((_CH,), jnp.int32),
        pltpu.VMEM((_CH,), jnp.int32),
        pltpu.VMEM((_CH,), jnp.float32),
        pltpu.VMEM((_CH, _D), jnp.float32),
        pltpu.SemaphoreType.DMA,
    ],
)
def _sc_spmm(h_hbm, idxj_hbm, idxi_hbm, vals_hbm, zeros_hbm, out_hbm,
             acc, idxj_v, idxi_v, vals_v, rows_v, sem):
    c = lax.axis_index("c")
    s = lax.axis_index("s")
    wid = c * 16 + s
    pltpu.sync_copy(zeros_hbm, acc.at[pl.ds(s * _ZROWS, _ZROWS)])
    plsc.subcore_barrier()

    ebase = wid * _EPT

    def chunk(ci, carry):
        off = ebase + ci * _CH
        pltpu.sync_copy(idxj_hbm.at[pl.ds(off, _CH)], idxj_v)
        pltpu.sync_copy(idxi_hbm.at[pl.ds(off, _CH)], idxi_v)
        pltpu.sync_copy(vals_hbm.at[pl.ds(off, _CH)], vals_v)
        pltpu.async_copy(h_hbm.at[idxj_v], rows_v, sem).wait()

        def row(r, carry2):
            v = vals_v[r]
            for g in range(_D // 16):
                sl = pl.ds(g * 16, 16)
                rows_v[r, sl] = rows_v[r, sl] * v
            return carry2

        lax.fori_loop(0, _CH, row, 0)
        pltpu.sync_copy(rows_v, acc.at[idxi_v], add=True)
        return carry

    lax.fori_loop(0, _NCHUNK, chunk, 0)
    plsc.subcore_barrier()
    pltpu.sync_copy(acc.at[pl.ds(s * _ZROWS, _ZROWS)],
                    out_hbm.at[c, pl.ds(s * _ZROWS, _ZROWS)])


@functools.partial(
    pl.kernel,
    out_type=jax.ShapeDtypeStruct((_G, _D), jnp.float32),
    mesh=_SC_MESH,
    scratch_types=[
        pltpu.VMEM((_GCH,), jnp.int32),
        pltpu.VMEM((_GCH, _D), jnp.float32),
        pltpu.SemaphoreType.DMA,
    ],
)
def _sc_gather(mat_hbm, idx_hbm, out_hbm, idx_v, rows_v, sem):
    c = lax.axis_index("c")
    s = lax.axis_index("s")
    wid = c * 16 + s
    gbase = wid * _GPT

    def chunk(ci, carry):
        off = gbase + ci * _GCH
        pltpu.sync_copy(idx_hbm.at[pl.ds(off, _GCH)], idx_v)
        pltpu.async_copy(mat_hbm.at[idx_v], rows_v, sem).wait()
        pltpu.sync_copy(rows_v, out_hbm.at[pl.ds(off, _GCH)])
        return carry

    lax.fori_loop(0, _GNCH, chunk, 0)


def _loss_body(s_ref, p_ref, n_ref, epw_ref, pw_ref, pb_ref, w0_ref, b0_ref,
               w1_ref, b1_ref, ow_ref, ob_ref, cl_ref, reg_ref):
    i = pl.program_id(0)

    @pl.when(i == 0)
    def _init():
        cl_ref[0, 0] = 0.0
        reg = jnp.zeros((), jnp.float32)
        for r in (epw_ref, pw_ref, pb_ref, w0_ref, b0_ref, w1_ref, b1_ref,
                  ow_ref, ob_ref):
            v = r[...]
            reg = reg + jnp.sum(v * v)
        reg_ref[0, 0] = reg * _LAMBDA1

    sm = s_ref[...]
    pm = p_ref[...]
    na = jnp.maximum(jnp.sqrt(jnp.sum(sm * sm, axis=1, keepdims=True)), 1e-8)
    nb = jnp.maximum(jnp.sqrt(jnp.sum(pm * pm, axis=1, keepdims=True)), 1e-8)
    ps = jnp.sum(sm * pm, axis=1, keepdims=True) / (na * nb)
    a = jnp.exp(ps / _TEMP)
    total = jnp.zeros((), jnp.float32)
    for k in range(_K):
        nk = n_ref[k]
        nn = jnp.maximum(jnp.sqrt(jnp.sum(nk * nk, axis=1, keepdims=True)),
                         1e-8)
        ns = jnp.sum(sm * nk, axis=1, keepdims=True) / (na * nn)
        bk = jnp.exp(ns / _TEMP)
        total = total + jnp.sum(-jnp.log(a / (a + bk + 1e-8)))
    cl_ref[0, 0] += total / (_B * _K)


def _loss(rows_s, rows_p, rows_n3, emb_p_w, proj_W, pb2, W0, b02, W1, b12,
          out_W, ob2):
    grid = (_B // _BB,)
    cl, reg = pl.pallas_call(
        _loss_body,
        grid=grid,
        in_specs=[
            pl.BlockSpec((_BB, _D), lambda i: (i, 0)),
            pl.BlockSpec((_BB, _D), lambda i: (i, 0)),
            pl.BlockSpec((_K, _BB, _D), lambda i: (0, i, 0)),
            pl.BlockSpec((_DEPTH, _PD), lambda i: (0, 0)),
            pl.BlockSpec((_D, _D + _PD), lambda i: (0, 0)),
            pl.BlockSpec((1, _D), lambda i: (0, 0)),
            pl.BlockSpec((_D, _D), lambda i: (0, 0)),
            pl.BlockSpec((1, _D), lambda i: (0, 0)),
            pl.BlockSpec((_D, _D), lambda i: (0, 0)),
            pl.BlockSpec((1, _D), lambda i: (0, 0)),
            pl.BlockSpec((_D, _D), lambda i: (0, 0)),
            pl.BlockSpec((1, _D), lambda i: (0, 0)),
        ],
        out_specs=[
            pl.BlockSpec((1, 1), lambda i: (0, 0)),
            pl.BlockSpec((1, 1), lambda i: (0, 0)),
        ],
        out_shape=[
            jax.ShapeDtypeStruct((1, 1), jnp.float32),
            jax.ShapeDtypeStruct((1, 1), jnp.float32),
        ],
    )(rows_s, rows_p, rows_n3, emb_p_w, proj_W, pb2, W0, b02, W1, b12,
      out_W, ob2)
    return cl, reg


def kernel(emb_s, edge_index, adj_values, position_ids, sids, pos, negs,
           emb_p_w, proj_W, proj_b, W0, b0, W1, b1, out_W, out_b):
    f32 = jnp.float32
    i32 = jnp.int32
    emb_s = emb_s.astype(f32)
    idx_i = edge_index[0].astype(i32)
    idx_j = edge_index[1].astype(i32)
    vals = adj_values.astype(f32)

    pad_e = _EPAD - _E
    idx_i_p = jnp.pad(idx_i, (0, pad_e))
    idx_j_p = jnp.pad(idx_j, (0, pad_e))
    vals_p = jnp.pad(vals, (0, pad_e))

    onehot = (position_ids.astype(i32)[:, None]
              == jnp.arange(_DEPTH, dtype=i32)[None, :]).astype(f32)
    zeros_blk = jnp.zeros((_ZROWS, _D), f32)

    pb2 = proj_b.reshape(1, _D).astype(f32)
    b02 = b0.reshape(1, _D).astype(f32)
    b12 = b1.reshape(1, _D).astype(f32)
    ob2 = out_b.reshape(1, _D).astype(f32)

    x0, h0 = _proj0(emb_s, onehot, emb_p_w.astype(f32), proj_W.astype(f32),
                    pb2, W0.astype(f32), b02)
    p = _sc_spmm(h0, idx_j_p, idx_i_p, vals_p, zeros_blk)
    x1, h1 = _residual_layer(x0, p[0, :_N], p[1, :_N], W1.astype(f32), b12,
                             want_xn=True)
    q = _sc_spmm(h1, idx_j_p, idx_i_p, vals_p, zeros_blk)
    out = _residual_layer(x1, q[0, :_N], q[1, :_N], out_W.astype(f32), ob2,
                          want_xn=False)

    gidx = jnp.concatenate([sids.astype(i32), pos.astype(i32),
                            negs.astype(i32).reshape(-1)])
    rows = _sc_gather(out, gidx)
    rows_s = rows[:_B]
    rows_p = rows[_B:2 * _B]
    rows_n3 = rows[2 * _B:].reshape(_K, _B, _D)

    cl, reg = _loss(rows_s, rows_p, rows_n3, emb_p_w.astype(f32),
                    proj_W.astype(f32), pb2, W0.astype(f32), b02,
                    W1.astype(f32), b12, out_W.astype(f32), ob2)
    loss_cl = cl[0, 0]
    loss_reg = reg[0, 0]
    return (loss_cl + loss_reg, loss_cl, loss_reg)


# speedup vs baseline: 2.5368x; 2.5368x over previous
import functools

import jax
import jax.numpy as jnp
from jax import lax
from jax.experimental import pallas as pl
from jax.experimental.pallas import tpu as pltpu
from jax.experimental.pallas import tpu_sc as plsc

_N = 10000
_E = 320000
_D = 128
_PD = 64
_DEPTH = 16
_B = 1024
_K = 32
_TEMP = 0.5
_LAMBDA1 = 1e-05

_NPAD = 10112
_EPT = 10240
_EPAD = 32 * _EPT
_CH = 128
_NCHUNK = _EPT // _CH
_ZROWS = _NPAD // 16
_G = _B * (_K + 2)
_GPT = _G // 32
_GCH = 64
_GNCH = _GPT // _GCH

_BN = 2000
_BB = 256


def _mm_nt(a, b):
    return lax.dot_general(a, b, (((1,), (1,)), ((), ())),
                           preferred_element_type=jnp.float32)


def _mm_nn(a, b):
    return lax.dot_general(a, b, (((1,), (0,)), ((), ())),
                           preferred_element_type=jnp.float32)


def _proj0_body(es_ref, oh_ref, epw_ref, pw_ref, pb_ref, w0_ref, b0_ref,
                x0_ref, h0_ref):
    pw = pw_ref[...]
    wa = pw[:, :_D]
    wb = pw[:, _D:]
    emb_pp = _mm_nt(epw_ref[...], wb)
    x0 = _mm_nt(es_ref[...], wa) + _mm_nn(oh_ref[...], emb_pp) + pb_ref[...]
    h0 = jnp.maximum(_mm_nt(x0, w0_ref[...]) + b0_ref[...], 0.0)
    x0_ref[...] = x0
    h0_ref[...] = h0


def _proj0(emb_s, onehot, emb_p_w, proj_W, pb2, W0, b02):
    grid = (_N // _BN,)
    return pl.pallas_call(
        _proj0_body,
        grid=grid,
        in_specs=[
            pl.BlockSpec((_BN, _D), lambda i: (i, 0)),
            pl.BlockSpec((_BN, _DEPTH), lambda i: (i, 0)),
            pl.BlockSpec((_DEPTH, _PD), lambda i: (0, 0)),
            pl.BlockSpec((_D, _D + _PD), lambda i: (0, 0)),
            pl.BlockSpec((1, _D), lambda i: (0, 0)),
            pl.BlockSpec((_D, _D), lambda i: (0, 0)),
            pl.BlockSpec((1, _D), lambda i: (0, 0)),
        ],
        out_specs=[
            pl.BlockSpec((_BN, _D), lambda i: (i, 0)),
            pl.BlockSpec((_BN, _D), lambda i: (i, 0)),
        ],
        out_shape=[
            jax.ShapeDtypeStruct((_N, _D), jnp.float32),
            jax.ShapeDtypeStruct((_N, _D), jnp.float32),
        ],
    )(emb_s, onehot, emb_p_w, proj_W, pb2, W0, b02)


def _res_body_both(x_ref, p0_ref, p1_ref, w_ref, b_ref, xn_ref, y_ref):
    xn = x_ref[...] + p0_ref[...] + p1_ref[...]
    y = jnp.maximum(_mm_nt(xn, w_ref[...]) + b_ref[...], 0.0)
    xn_ref[...] = xn
    y_ref[...] = y


def _res_body_out(x_ref, p0_ref, p1_ref, w_ref, b_ref, y_ref):
    xn = x_ref[...] + p0_ref[...] + p1_ref[...]
    y_ref[...] = _mm_nt(xn, w_ref[...]) + b_ref[...]


def _residual_layer(x, p0, p1, W, b2, want_xn):
    grid = (_N // _BN,)
    in_specs = [
        pl.BlockSpec((_BN, _D), lambda i: (i, 0)),
        pl.BlockSpec((_BN, _D), lambda i: (i, 0)),
        pl.BlockSpec((_BN, _D), lambda i: (i, 0)),
        pl.BlockSpec((_D, _D), lambda i: (0, 0)),
        pl.BlockSpec((1, _D), lambda i: (0, 0)),
    ]
    if want_xn:
        return pl.pallas_call(
            _res_body_both,
            grid=grid,
            in_specs=in_specs,
            out_specs=[pl.BlockSpec((_BN, _D), lambda i: (i, 0))] * 2,
            out_shape=[jax.ShapeDtypeStruct((_N, _D), jnp.float32)] * 2,
        )(x, p0, p1, W, b2)
    return pl.pallas_call(
        _res_body_out,
        grid=grid,
        in_specs=in_specs,
        out_specs=pl.BlockSpec((_BN, _D), lambda i: (i, 0)),
        out_shape=jax.ShapeDtypeStruct((_N, _D), jnp.float32),
    )(x, p0, p1, W, b2)


_SC_MESH = plsc.VectorSubcoreMesh(core_axis_name="c", subcore_axis_name="s")


@functools.partial(
    pl.kernel,
    out_type=jax.ShapeDtypeStruct((2, _NPAD, _D), jnp.float32),
    mesh=_SC_MESH,
    scratch_types=[
        pltpu.VMEM_SHARED((_NPAD, _D), jnp.float32),
        pltpu.VMEM((_CH,), jnp.int32),
        pltpu.VMEM((_CH,), jnp.int32),
        pltpu.VMEM((_CH,), jnp.float32),
        pltpu.VMEM((_CH, _D), jnp.float32),
        pltpu.SemaphoreType.DMA,
    ],
)
def _sc_spmm(h_hbm, idxj_hbm, idxi_hbm, vals_hbm, zeros_hbm, out_hbm,
             acc, idxj_v, idxi_v, vals_v, rows_v, sem):
    c = lax.axis_index("c")
    s = lax.axis_index("s")
    wid = c * 16 + s
    pltpu.sync_copy(zeros_hbm, acc.at[pl.ds(s * _ZROWS, _ZROWS)])
    plsc.subcore_barrier()

    ebase = wid * _EPT

    def chunk(ci, carry):
        off = ebase + ci * _CH
        pltpu.sync_copy(idxj_hbm.at[pl.ds(off, _CH)], idxj_v)
        pltpu.sync_copy(idxi_hbm.at[pl.ds(off, _CH)], idxi_v)
        pltpu.sync_copy(vals_hbm.at[pl.ds(off, _CH)], vals_v)
        pltpu.async_copy(h_hbm.at[idxj_v], rows_v, sem).wait()

        def grp(t, carry2):
            vseg = vals_v[pl.ds(t * 16, 16)]
            for l in range(16):
                v = vseg[l]
                r = t * 16 + l
                for g in range(_D // 16):
                    sl = pl.ds(g * 16, 16)
                    rows_v[r, sl] = rows_v[r, sl] * v
            return carry2

        lax.fori_loop(0, _CH // 16, grp, 0)
        pltpu.sync_copy(rows_v, acc.at[idxi_v], add=True)
        return carry

    lax.fori_loop(0, _NCHUNK, chunk, 0)
    plsc.subcore_barrier()
    pltpu.sync_copy(acc.at[pl.ds(s * _ZROWS, _ZROWS)],
                    out_hbm.at[c, pl.ds(s * _ZROWS, _ZROWS)])


@functools.partial(
    pl.kernel,
    out_type=jax.ShapeDtypeStruct((_G, _D), jnp.float32),
    mesh=_SC_MESH,
    scratch_types=[
        pltpu.VMEM((_GCH,), jnp.int32),
        pltpu.VMEM((_GCH, _D), jnp.float32),
        pltpu.SemaphoreType.DMA,
    ],
)
def _sc_gather(mat_hbm, idx_hbm, out_hbm, idx_v, rows_v, sem):
    c = lax.axis_index("c")
    s = lax.axis_index("s")
    wid = c * 16 + s
    gbase = wid * _GPT

    def chunk(ci, carry):
        off = gbase + ci * _GCH
        pltpu.sync_copy(idx_hbm.at[pl.ds(off, _GCH)], idx_v)
        pltpu.async_copy(mat_hbm.at[idx_v], rows_v, sem).wait()
        pltpu.sync_copy(rows_v, out_hbm.at[pl.ds(off, _GCH)])
        return carry

    lax.fori_loop(0, _GNCH, chunk, 0)


def _loss_body(s_ref, p_ref, n_ref, epw_ref, pw_ref, pb_ref, w0_ref, b0_ref,
               w1_ref, b1_ref, ow_ref, ob_ref, cl_ref, reg_ref):
    i = pl.program_id(0)

    @pl.when(i == 0)
    def _init():
        cl_ref[...] = jnp.zeros((1, 1), jnp.float32)
        reg = jnp.zeros((), jnp.float32)
        for r in (epw_ref, pw_ref, pb_ref, w0_ref, b0_ref, w1_ref, b1_ref,
                  ow_ref, ob_ref):
            v = r[...]
            reg = reg + jnp.sum(v * v)
        reg_ref[...] = jnp.reshape(reg * _LAMBDA1, (1, 1))

    sm = s_ref[...]
    pm = p_ref[...]
    na = jnp.maximum(jnp.sqrt(jnp.sum(sm * sm, axis=1, keepdims=True)), 1e-8)
    nb = jnp.maximum(jnp.sqrt(jnp.sum(pm * pm, axis=1, keepdims=True)), 1e-8)
    ps = jnp.sum(sm * pm, axis=1, keepdims=True) / (na * nb)
    a = jnp.exp(ps / _TEMP)
    total = jnp.zeros((), jnp.float32)
    for k in range(_K):
        nk = n_ref[k]
        nn = jnp.maximum(jnp.sqrt(jnp.sum(nk * nk, axis=1, keepdims=True)),
                         1e-8)
        ns = jnp.sum(sm * nk, axis=1, keepdims=True) / (na * nn)
        bk = jnp.exp(ns / _TEMP)
        total = total + jnp.sum(-jnp.log(a / (a + bk + 1e-8)))
    cl_ref[...] = cl_ref[...] + jnp.reshape(total / (_B * _K), (1, 1))


def _loss(rows_s, rows_p, rows_n3, emb_p_w, proj_W, pb2, W0, b02, W1, b12,
          out_W, ob2):
    grid = (_B // _BB,)
    cl, reg = pl.pallas_call(
        _loss_body,
        grid=grid,
        in_specs=[
            pl.BlockSpec((_BB, _D), lambda i: (i, 0)),
            pl.BlockSpec((_BB, _D), lambda i: (i, 0)),
            pl.BlockSpec((_K, _BB, _D), lambda i: (0, i, 0)),
            pl.BlockSpec((_DEPTH, _PD), lambda i: (0, 0)),
            pl.BlockSpec((_D, _D + _PD), lambda i: (0, 0)),
            pl.BlockSpec((1, _D), lambda i: (0, 0)),
            pl.BlockSpec((_D, _D), lambda i: (0, 0)),
            pl.BlockSpec((1, _D), lambda i: (0, 0)),
            pl.BlockSpec((_D, _D), lambda i: (0, 0)),
            pl.BlockSpec((1, _D), lambda i: (0, 0)),
            pl.BlockSpec((_D, _D), lambda i: (0, 0)),
            pl.BlockSpec((1, _D), lambda i: (0, 0)),
        ],
        out_specs=[
            pl.BlockSpec((1, 1), lambda i: (0, 0)),
            pl.BlockSpec((1, 1), lambda i: (0, 0)),
        ],
        out_shape=[
            jax.ShapeDtypeStruct((1, 1), jnp.float32),
            jax.ShapeDtypeStruct((1, 1), jnp.float32),
        ],
    )(rows_s, rows_p, rows_n3, emb_p_w, proj_W, pb2, W0, b02, W1, b12,
      out_W, ob2)
    return cl, reg


def kernel(emb_s, edge_index, adj_values, position_ids, sids, pos, negs,
           emb_p_w, proj_W, proj_b, W0, b0, W1, b1, out_W, out_b):
    f32 = jnp.float32
    i32 = jnp.int32
    emb_s = emb_s.astype(f32)
    idx_i = edge_index[0].astype(i32)
    idx_j = edge_index[1].astype(i32)
    vals = adj_values.astype(f32)

    pad_e = _EPAD - _E
    idx_i_p = jnp.pad(idx_i, (0, pad_e))
    idx_j_p = jnp.pad(idx_j, (0, pad_e))
    vals_p = jnp.pad(vals, (0, pad_e))

    onehot = (position_ids.astype(i32)[:, None]
              == jnp.arange(_DEPTH, dtype=i32)[None, :]).astype(f32)
    zeros_blk = jnp.zeros((_ZROWS, _D), f32)

    pb2 = proj_b.reshape(1, _D).astype(f32)
    b02 = b0.reshape(1, _D).astype(f32)
    b12 = b1.reshape(1, _D).astype(f32)
    ob2 = out_b.reshape(1, _D).astype(f32)

    x0, h0 = _proj0(emb_s, onehot, emb_p_w.astype(f32), proj_W.astype(f32),
                    pb2, W0.astype(f32), b02)
    p = _sc_spmm(h0, idx_j_p, idx_i_p, vals_p, zeros_blk)
    x1, h1 = _residual_layer(x0, p[0, :_N], p[1, :_N], W1.astype(f32), b12,
                             want_xn=True)
    q = _sc_spmm(h1, idx_j_p, idx_i_p, vals_p, zeros_blk)
    out = _residual_layer(x1, q[0, :_N], q[1, :_N], out_W.astype(f32), ob2,
                          want_xn=False)

    gidx = jnp.concatenate([sids.astype(i32), pos.astype(i32),
                            negs.astype(i32).reshape(-1)])
    rows = _sc_gather(out, gidx)
    rows_s = rows[:_B]
    rows_p = rows[_B:2 * _B]
    rows_n3 = rows[2 * _B:].reshape(_K, _B, _D)

    cl, reg = _loss(rows_s, rows_p, rows_n3, emb_p_w.astype(f32),
                    proj_W.astype(f32), pb2, W0.astype(f32), b02,
                    W1.astype(f32), b12, out_W.astype(f32), ob2)
    loss_cl = cl[0, 0]
    loss_reg = reg[0, 0]
    return (loss_cl + loss_reg, loss_cl, loss_reg)


# speedup vs baseline: 3.1683x; 1.2489x over previous
import functools

import jax
import jax.numpy as jnp
from jax import lax
from jax.experimental import pallas as pl
from jax.experimental.pallas import tpu as pltpu
from jax.experimental.pallas import tpu_sc as plsc

_N = 10000
_E = 320000
_D = 128
_PD = 64
_DEPTH = 16
_B = 1024
_K = 32
_TEMP = 0.5
_LAMBDA1 = 1e-05

_NPAD = 10112
_EPT = 10240
_EPAD = 32 * _EPT
_CH = 64
_NCHUNK = _EPT // _CH
_ZROWS = _NPAD // 16
_G = _B * (_K + 2)
_GPT = _G // 32
_GCH = 64
_GNCH = _GPT // _GCH

_BN = 2000
_BB = 256


def _mm_nt(a, b):
    return lax.dot_general(a, b, (((1,), (1,)), ((), ())),
                           preferred_element_type=jnp.float32)


def _mm_nn(a, b):
    return lax.dot_general(a, b, (((1,), (0,)), ((), ())),
                           preferred_element_type=jnp.float32)


def _proj0_body(es_ref, oh_ref, epw_ref, pw_ref, pb_ref, w0_ref, b0_ref,
                x0_ref, h0_ref):
    pw = pw_ref[...]
    wa = pw[:, :_D]
    wb = pw[:, _D:]
    emb_pp = _mm_nt(epw_ref[...], wb)
    x0 = _mm_nt(es_ref[...], wa) + _mm_nn(oh_ref[...], emb_pp) + pb_ref[...]
    h0 = jnp.maximum(_mm_nt(x0, w0_ref[...]) + b0_ref[...], 0.0)
    x0_ref[...] = x0
    h0_ref[...] = h0


def _proj0(emb_s, onehot, emb_p_w, proj_W, pb2, W0, b02):
    grid = (_N // _BN,)
    return pl.pallas_call(
        _proj0_body,
        grid=grid,
        in_specs=[
            pl.BlockSpec((_BN, _D), lambda i: (i, 0)),
            pl.BlockSpec((_BN, _DEPTH), lambda i: (i, 0)),
            pl.BlockSpec((_DEPTH, _PD), lambda i: (0, 0)),
            pl.BlockSpec((_D, _D + _PD), lambda i: (0, 0)),
            pl.BlockSpec((1, _D), lambda i: (0, 0)),
            pl.BlockSpec((_D, _D), lambda i: (0, 0)),
            pl.BlockSpec((1, _D), lambda i: (0, 0)),
        ],
        out_specs=[
            pl.BlockSpec((_BN, _D), lambda i: (i, 0)),
            pl.BlockSpec((_BN, _D), lambda i: (i, 0)),
        ],
        out_shape=[
            jax.ShapeDtypeStruct((_N, _D), jnp.float32),
            jax.ShapeDtypeStruct((_N, _D), jnp.float32),
        ],
    )(emb_s, onehot, emb_p_w, proj_W, pb2, W0, b02)


def _res_body_both(x_ref, p0_ref, p1_ref, w_ref, b_ref, xn_ref, y_ref):
    xn = x_ref[...] + p0_ref[...] + p1_ref[...]
    y = jnp.maximum(_mm_nt(xn, w_ref[...]) + b_ref[...], 0.0)
    xn_ref[...] = xn
    y_ref[...] = y


def _res_body_out(x_ref, p0_ref, p1_ref, w_ref, b_ref, y_ref):
    xn = x_ref[...] + p0_ref[...] + p1_ref[...]
    y_ref[...] = _mm_nt(xn, w_ref[...]) + b_ref[...]


def _residual_layer(x, p0, p1, W, b2, want_xn):
    grid = (_N // _BN,)
    in_specs = [
        pl.BlockSpec((_BN, _D), lambda i: (i, 0)),
        pl.BlockSpec((_BN, _D), lambda i: (i, 0)),
        pl.BlockSpec((_BN, _D), lambda i: (i, 0)),
        pl.BlockSpec((_D, _D), lambda i: (0, 0)),
        pl.BlockSpec((1, _D), lambda i: (0, 0)),
    ]
    if want_xn:
        return pl.pallas_call(
            _res_body_both,
            grid=grid,
            in_specs=in_specs,
            out_specs=[pl.BlockSpec((_BN, _D), lambda i: (i, 0))] * 2,
            out_shape=[jax.ShapeDtypeStruct((_N, _D), jnp.float32)] * 2,
        )(x, p0, p1, W, b2)
    return pl.pallas_call(
        _res_body_out,
        grid=grid,
        in_specs=in_specs,
        out_specs=pl.BlockSpec((_BN, _D), lambda i: (i, 0)),
        out_shape=jax.ShapeDtypeStruct((_N, _D), jnp.float32),
    )(x, p0, p1, W, b2)


_SC_MESH = plsc.VectorSubcoreMesh(core_axis_name="c", subcore_axis_name="s")

_MW = 3 * _CH


@functools.partial(
    pl.kernel,
    out_type=jax.ShapeDtypeStruct((2, _NPAD, _D), jnp.float32),
    mesh=_SC_MESH,
    scratch_types=(
        [pltpu.VMEM_SHARED((_NPAD, _D), jnp.float32)]
        + [pltpu.VMEM((_MW,), jnp.int32)] * 4
        + [pltpu.VMEM((_CH, _D), jnp.float32)] * 4
        + [pltpu.VMEM((_CH,), jnp.int32)] * 4
        + [pltpu.SemaphoreType.DMA] * 12
    ),
)
def _sc_spmm(h_hbm, meta_hbm, zeros_hbm, out_hbm,
             acc, m0, m1, m2, m3, r0, r1, r2, r3, i0, i1, i2, i3,
             sm0, sm1, sm2, sm3, sg0, sg1, sg2, sg3, ss0, ss1, ss2, ss3):
    c = lax.axis_index("c")
    s = lax.axis_index("s")
    wid = c * 16 + s
    pltpu.sync_copy(zeros_hbm, acc.at[pl.ds(s * _ZROWS, _ZROWS)])
    plsc.subcore_barrier()

    base = wid * _NCHUNK
    bufs = (
        (m0, r0, i0, sm0, sg0, ss0),
        (m1, r1, i1, sm1, sg1, ss1),
        (m2, r2, i2, sm2, sg2, ss2),
        (m3, r3, i3, sm3, sg3, ss3),
    )

    def start_meta(blk, buf):
        pltpu.async_copy(meta_hbm.at[pl.ds(blk * _MW, _MW)], buf[0], buf[3])

    def wait_meta(buf):
        pltpu.make_async_copy(meta_hbm.at[pl.ds(0, _MW)], buf[0],
                              buf[3]).wait()

    def start_gather(buf):
        pltpu.async_copy(h_hbm.at[buf[0].at[pl.ds(0, _CH)]], buf[1], buf[4])

    def wait_gather(buf):
        pltpu.make_async_copy(h_hbm.at[buf[0].at[pl.ds(0, _CH)]], buf[1],
                              buf[4]).wait()

    def start_scatter(buf):
        pltpu.async_copy(buf[1], acc.at[buf[2]], buf[5], add=True)

    def wait_scatter(buf):
        pltpu.make_async_copy(buf[1], acc.at[buf[2]], buf[5]).wait()

    def scale(buf):
        mv, rv = buf[0], buf[1]

        def grp(t, carry2):
            vseg = mv[pl.ds(2 * _CH + t * 16, 16)]
            for l in range(16):
                v = lax.bitcast_convert_type(vseg[l], jnp.float32)
                r = t * 16 + l
                for g in range(_D // 16):
                    sl = pl.ds(g * 16, 16)
                    rv[r, sl] = rv[r, sl] * v
            return carry2

        lax.fori_loop(0, _CH // 16, grp, 0)

    def copy_idxi(buf):
        mv, iv = buf[0], buf[2]
        for t in range(_CH // 16):
            iv[pl.ds(t * 16, 16)] = mv[pl.ds(_CH + t * 16, 16)]

    for b in range(4):
        start_meta(base + b, bufs[b])
    for b in range(2):
        wait_meta(bufs[b])
        start_gather(bufs[b])

    def quarter(u, off):
        ci = 4 * u + off
        p = bufs[off]
        nxt = bufs[(off + 2) % 4]
        wait_gather(p)

        @pl.when(ci + 2 < _NCHUNK)
        def _():
            wait_meta(nxt)

            @pl.when(ci >= 2)
            def _():
                wait_scatter(nxt)

            start_gather(nxt)

        scale(p)
        copy_idxi(p)
        start_scatter(p)

        @pl.when(ci + 4 < _NCHUNK)
        def _():
            start_meta(base + ci + 4, p)

    def body4(u, carry):
        for off in range(4):
            quarter(u, off)
        return carry

    lax.fori_loop(0, _NCHUNK // 4, body4, 0)
    for b in range(4):
        wait_scatter(bufs[b])
    plsc.subcore_barrier()
    pltpu.sync_copy(acc.at[pl.ds(s * _ZROWS, _ZROWS)],
                    out_hbm.at[c, pl.ds(s * _ZROWS, _ZROWS)])


@functools.partial(
    pl.kernel,
    out_type=jax.ShapeDtypeStruct((_G, _D), jnp.float32),
    mesh=_SC_MESH,
    scratch_types=[
        pltpu.VMEM((_GCH,), jnp.int32),
        pltpu.VMEM((_GCH, _D), jnp.float32),
        pltpu.SemaphoreType.DMA,
    ],
)
def _sc_gather(mat_hbm, idx_hbm, out_hbm, idx_v, rows_v, sem):
    c = lax.axis_index("c")
    s = lax.axis_index("s")
    wid = c * 16 + s
    gbase = wid * _GPT

    def chunk(ci, carry):
        off = gbase + ci * _GCH
        pltpu.sync_copy(idx_hbm.at[pl.ds(off, _GCH)], idx_v)
        pltpu.async_copy(mat_hbm.at[idx_v], rows_v, sem).wait()
        pltpu.sync_copy(rows_v, out_hbm.at[pl.ds(off, _GCH)])
        return carry

    lax.fori_loop(0, _GNCH, chunk, 0)


def _loss_body(s_ref, p_ref, n_ref, epw_ref, pw_ref, pb_ref, w0_ref, b0_ref,
               w1_ref, b1_ref, ow_ref, ob_ref, cl_ref, reg_ref):
    i = pl.program_id(0)

    @pl.when(i == 0)
    def _init():
        cl_ref[...] = jnp.zeros((1, 1), jnp.float32)
        reg = jnp.zeros((), jnp.float32)
        for r in (epw_ref, pw_ref, pb_ref, w0_ref, b0_ref, w1_ref, b1_ref,
                  ow_ref, ob_ref):
            v = r[...]
            reg = reg + jnp.sum(v * v)
        reg_ref[...] = jnp.reshape(reg * _LAMBDA1, (1, 1))

    sm = s_ref[...]
    pm = p_ref[...]
    na = jnp.maximum(jnp.sqrt(jnp.sum(sm * sm, axis=1, keepdims=True)), 1e-8)
    nb = jnp.maximum(jnp.sqrt(jnp.sum(pm * pm, axis=1, keepdims=True)), 1e-8)
    ps = jnp.sum(sm * pm, axis=1, keepdims=True) / (na * nb)
    a = jnp.exp(ps / _TEMP)
    total = jnp.zeros((), jnp.float32)
    for k in range(_K):
        nk = n_ref[k]
        nn = jnp.maximum(jnp.sqrt(jnp.sum(nk * nk, axis=1, keepdims=True)),
                         1e-8)
        ns = jnp.sum(sm * nk, axis=1, keepdims=True) / (na * nn)
        bk = jnp.exp(ns / _TEMP)
        total = total + jnp.sum(-jnp.log(a / (a + bk + 1e-8)))
    cl_ref[...] = cl_ref[...] + jnp.reshape(total / (_B * _K), (1, 1))


def _loss(rows_s, rows_p, rows_n3, emb_p_w, proj_W, pb2, W0, b02, W1, b12,
          out_W, ob2):
    grid = (_B // _BB,)
    cl, reg = pl.pallas_call(
        _loss_body,
        grid=grid,
        in_specs=[
            pl.BlockSpec((_BB, _D), lambda i: (i, 0)),
            pl.BlockSpec((_BB, _D), lambda i: (i, 0)),
            pl.BlockSpec((_K, _BB, _D), lambda i: (0, i, 0)),
            pl.BlockSpec((_DEPTH, _PD), lambda i: (0, 0)),
            pl.BlockSpec((_D, _D + _PD), lambda i: (0, 0)),
            pl.BlockSpec((1, _D), lambda i: (0, 0)),
            pl.BlockSpec((_D, _D), lambda i: (0, 0)),
            pl.BlockSpec((1, _D), lambda i: (0, 0)),
            pl.BlockSpec((_D, _D), lambda i: (0, 0)),
            pl.BlockSpec((1, _D), lambda i: (0, 0)),
            pl.BlockSpec((_D, _D), lambda i: (0, 0)),
            pl.BlockSpec((1, _D), lambda i: (0, 0)),
        ],
        out_specs=[
            pl.BlockSpec((1, 1), lambda i: (0, 0)),
            pl.BlockSpec((1, 1), lambda i: (0, 0)),
        ],
        out_shape=[
            jax.ShapeDtypeStruct((1, 1), jnp.float32),
            jax.ShapeDtypeStruct((1, 1), jnp.float32),
        ],
    )(rows_s, rows_p, rows_n3, emb_p_w, proj_W, pb2, W0, b02, W1, b12,
      out_W, ob2)
    return cl, reg


def kernel(emb_s, edge_index, adj_values, position_ids, sids, pos, negs,
           emb_p_w, proj_W, proj_b, W0, b0, W1, b1, out_W, out_b):
    f32 = jnp.float32
    i32 = jnp.int32
    emb_s = emb_s.astype(f32)
    idx_i = edge_index[0].astype(i32)
    idx_j = edge_index[1].astype(i32)
    vals = adj_values.astype(f32)

    pad_e = _EPAD - _E
    idx_i_p = jnp.pad(idx_i, (0, pad_e))
    idx_j_p = jnp.pad(idx_j, (0, pad_e))
    vals_p = jnp.pad(vals, (0, pad_e))
    nblk = _EPAD // _CH
    meta = jnp.stack(
        [idx_j_p.reshape(nblk, _CH), idx_i_p.reshape(nblk, _CH),
         lax.bitcast_convert_type(vals_p, i32).reshape(nblk, _CH)],
        axis=1).reshape(-1)

    onehot = (position_ids.astype(i32)[:, None]
              == jnp.arange(_DEPTH, dtype=i32)[None, :]).astype(f32)
    zeros_blk = jnp.zeros((_ZROWS, _D), f32)

    pb2 = proj_b.reshape(1, _D).astype(f32)
    b02 = b0.reshape(1, _D).astype(f32)
    b12 = b1.reshape(1, _D).astype(f32)
    ob2 = out_b.reshape(1, _D).astype(f32)

    x0, h0 = _proj0(emb_s, onehot, emb_p_w.astype(f32), proj_W.astype(f32),
                    pb2, W0.astype(f32), b02)
    p = _sc_spmm(h0, meta, zeros_blk)
    x1, h1 = _residual_layer(x0, p[0, :_N], p[1, :_N], W1.astype(f32), b12,
                             want_xn=True)
    q = _sc_spmm(h1, meta, zeros_blk)
    out = _residual_layer(x1, q[0, :_N], q[1, :_N], out_W.astype(f32), ob2,
                          want_xn=False)

    gidx = jnp.concatenate([sids.astype(i32), pos.astype(i32),
                            negs.astype(i32).reshape(-1)])
    rows = _sc_gather(out, gidx)
    rows_s = rows[:_B]
    rows_p = rows[_B:2 * _B]
    rows_n3 = rows[2 * _B:].reshape(_K, _B, _D)

    cl, reg = _loss(rows_s, rows_p, rows_n3, emb_p_w.astype(f32),
                    proj_W.astype(f32), pb2, W0.astype(f32), b02,
                    W1.astype(f32), b12, out_W.astype(f32), ob2)
    loss_cl = cl[0, 0]
    loss_reg = reg[0, 0]
    return (loss_cl + loss_reg, loss_cl, loss_reg)


# speedup vs baseline: 3.1954x; 1.0086x over previous
import functools

import jax
import jax.numpy as jnp
from jax import lax
from jax.experimental import pallas as pl
from jax.experimental.pallas import tpu as pltpu
from jax.experimental.pallas import tpu_sc as plsc

_N = 10000
_E = 320000
_D = 128
_PD = 64
_DEPTH = 16
_B = 1024
_K = 32
_TEMP = 0.5
_LAMBDA1 = 1e-05

_NPAD = 10112
_EPT = 10240
_EPAD = 32 * _EPT
_CH = 64
_NCHUNK = _EPT // _CH
_ZROWS = _NPAD // 16
_G = _B * (_K + 2)
_GPT = _G // 32
_GCH = 64
_GNCH = _GPT // _GCH

_BN = 2000
_BB = 256


def _mm_nt(a, b):
    return lax.dot_general(a, b, (((1,), (1,)), ((), ())),
                           preferred_element_type=jnp.float32)


def _mm_nn(a, b):
    return lax.dot_general(a, b, (((1,), (0,)), ((), ())),
                           preferred_element_type=jnp.float32)


def _proj0_body(es_ref, oh_ref, epw_ref, pw_ref, pb_ref, w0_ref, b0_ref,
                x0_ref, h0_ref):
    pw = pw_ref[...]
    wa = pw[:, :_D]
    wb = pw[:, _D:]
    emb_pp = _mm_nt(epw_ref[...], wb)
    x0 = _mm_nt(es_ref[...], wa) + _mm_nn(oh_ref[...], emb_pp) + pb_ref[...]
    h0 = jnp.maximum(_mm_nt(x0, w0_ref[...]) + b0_ref[...], 0.0)
    x0_ref[...] = x0
    h0_ref[...] = h0


def _proj0(emb_s, onehot, emb_p_w, proj_W, pb2, W0, b02):
    grid = (_N // _BN,)
    return pl.pallas_call(
        _proj0_body,
        grid=grid,
        in_specs=[
            pl.BlockSpec((_BN, _D), lambda i: (i, 0)),
            pl.BlockSpec((_BN, _DEPTH), lambda i: (i, 0)),
            pl.BlockSpec((_DEPTH, _PD), lambda i: (0, 0)),
            pl.BlockSpec((_D, _D + _PD), lambda i: (0, 0)),
            pl.BlockSpec((1, _D), lambda i: (0, 0)),
            pl.BlockSpec((_D, _D), lambda i: (0, 0)),
            pl.BlockSpec((1, _D), lambda i: (0, 0)),
        ],
        out_specs=[
            pl.BlockSpec((_BN, _D), lambda i: (i, 0)),
            pl.BlockSpec((_BN, _D), lambda i: (i, 0)),
        ],
        out_shape=[
            jax.ShapeDtypeStruct((_N, _D), jnp.float32),
            jax.ShapeDtypeStruct((_N, _D), jnp.float32),
        ],
    )(emb_s, onehot, emb_p_w, proj_W, pb2, W0, b02)


def _res_body_both(x_ref, p0_ref, p1_ref, w_ref, b_ref, xn_ref, y_ref):
    xn = x_ref[...] + p0_ref[...] + p1_ref[...]
    y = jnp.maximum(_mm_nt(xn, w_ref[...]) + b_ref[...], 0.0)
    xn_ref[...] = xn
    y_ref[...] = y


def _res_body_out(x_ref, p0_ref, p1_ref, w_ref, b_ref, y_ref):
    xn = x_ref[...] + p0_ref[...] + p1_ref[...]
    y_ref[...] = _mm_nt(xn, w_ref[...]) + b_ref[...]


def _residual_layer(x, p0, p1, W, b2, want_xn):
    grid = (_N // _BN,)
    in_specs = [
        pl.BlockSpec((_BN, _D), lambda i: (i, 0)),
        pl.BlockSpec((_BN, _D), lambda i: (i, 0)),
        pl.BlockSpec((_BN, _D), lambda i: (i, 0)),
        pl.BlockSpec((_D, _D), lambda i: (0, 0)),
        pl.BlockSpec((1, _D), lambda i: (0, 0)),
    ]
    if want_xn:
        return pl.pallas_call(
            _res_body_both,
            grid=grid,
            in_specs=in_specs,
            out_specs=[pl.BlockSpec((_BN, _D), lambda i: (i, 0))] * 2,
            out_shape=[jax.ShapeDtypeStruct((_N, _D), jnp.float32)] * 2,
        )(x, p0, p1, W, b2)
    return pl.pallas_call(
        _res_body_out,
        grid=grid,
        in_specs=in_specs,
        out_specs=pl.BlockSpec((_BN, _D), lambda i: (i, 0)),
        out_shape=jax.ShapeDtypeStruct((_N, _D), jnp.float32),
    )(x, p0, p1, W, b2)


_SC_MESH = plsc.VectorSubcoreMesh(core_axis_name="c", subcore_axis_name="s")

_MW = 3 * _CH


@functools.partial(
    pl.kernel,
    out_type=jax.ShapeDtypeStruct((2, _NPAD, _D), jnp.float32),
    mesh=_SC_MESH,
    scratch_types=(
        [pltpu.VMEM_SHARED((_NPAD, _D), jnp.float32)]
        + [pltpu.VMEM((_MW,), jnp.int32)] * 4
        + [pltpu.VMEM((_CH, _D), jnp.float32)] * 4
        + [pltpu.VMEM((_CH,), jnp.int32)] * 4
        + [pltpu.SemaphoreType.DMA] * 12
    ),
)
def _sc_spmm(h_hbm, meta_hbm, zeros_hbm, out_hbm,
             acc, m0, m1, m2, m3, r0, r1, r2, r3, i0, i1, i2, i3,
             sm0, sm1, sm2, sm3, sg0, sg1, sg2, sg3, ss0, ss1, ss2, ss3):
    c = lax.axis_index("c")
    s = lax.axis_index("s")
    wid = c * 16 + s
    pltpu.sync_copy(zeros_hbm, acc.at[pl.ds(s * _ZROWS, _ZROWS)])
    plsc.subcore_barrier()

    base = wid * _NCHUNK
    bufs = (
        (m0, r0, i0, sm0, sg0, ss0),
        (m1, r1, i1, sm1, sg1, ss1),
        (m2, r2, i2, sm2, sg2, ss2),
        (m3, r3, i3, sm3, sg3, ss3),
    )

    def start_meta(blk, buf):
        pltpu.async_copy(meta_hbm.at[pl.ds(blk * _MW, _MW)], buf[0], buf[3])

    def wait_meta(buf):
        pltpu.make_async_copy(meta_hbm.at[pl.ds(0, _MW)], buf[0],
                              buf[3]).wait()

    def start_gather(buf):
        pltpu.async_copy(h_hbm.at[buf[0].at[pl.ds(0, _CH)]], buf[1], buf[4])

    def wait_gather(buf):
        pltpu.make_async_copy(h_hbm.at[buf[0].at[pl.ds(0, _CH)]], buf[1],
                              buf[4]).wait()

    def start_scatter(buf):
        pltpu.async_copy(buf[1], acc.at[buf[2]], buf[5], add=True)

    def wait_scatter(buf):
        pltpu.make_async_copy(buf[1], acc.at[buf[2]], buf[5]).wait()

    def scale(buf):
        mv, rv = buf[0], buf[1]

        def grp(t, carry2):
            vseg = mv[pl.ds(2 * _CH + t * 16, 16)]
            for l in range(16):
                v = lax.bitcast_convert_type(vseg[l], jnp.float32)
                r = t * 16 + l
                for g in range(_D // 16):
                    sl = pl.ds(g * 16, 16)
                    rv[r, sl] = rv[r, sl] * v
            return carry2

        lax.fori_loop(0, _CH // 16, grp, 0)

    def copy_idxi(buf):
        mv, iv = buf[0], buf[2]
        for t in range(_CH // 16):
            iv[pl.ds(t * 16, 16)] = mv[pl.ds(_CH + t * 16, 16)]

    for b in range(4):
        start_meta(base + b, bufs[b])
    for b in range(2):
        wait_meta(bufs[b])
        start_gather(bufs[b])

    def quarter(u, off):
        ci = 4 * u + off
        p = bufs[off]
        nxt = bufs[(off + 2) % 4]
        wait_gather(p)

        @pl.when(ci + 2 < _NCHUNK)
        def _():
            wait_meta(nxt)

            @pl.when(ci >= 2)
            def _():
                wait_scatter(nxt)

            start_gather(nxt)

        scale(p)
        copy_idxi(p)
        start_scatter(p)

        @pl.when(ci + 4 < _NCHUNK)
        def _():
            start_meta(base + ci + 4, p)

    def body4(u, carry):
        for off in range(4):
            quarter(u, off)
        return carry

    lax.fori_loop(0, _NCHUNK // 4, body4, 0)
    for b in range(4):
        wait_scatter(bufs[b])
    plsc.subcore_barrier()
    pltpu.sync_copy(acc.at[pl.ds(s * _ZROWS, _ZROWS)],
                    out_hbm.at[c, pl.ds(s * _ZROWS, _ZROWS)])


@functools.partial(
    pl.kernel,
    out_type=jax.ShapeDtypeStruct((_G, _D), jnp.float32),
    mesh=_SC_MESH,
    scratch_types=(
        [pltpu.VMEM((_GCH,), jnp.int32)] * 2
        + [pltpu.VMEM((_GCH, _D), jnp.float32)] * 2
        + [pltpu.SemaphoreType.DMA] * 6
    ),
)
def _sc_gather(mat_hbm, idx_hbm, out_hbm, ia, ib, ra, rb,
               sia, sib, sga, sgb, swa, swb):
    c = lax.axis_index("c")
    s = lax.axis_index("s")
    wid = c * 16 + s
    gbase = wid * _GPT
    bufs = ((ia, ra, sia, sga, swa), (ib, rb, sib, sgb, swb))

    def start_idx(ci, buf):
        pltpu.async_copy(idx_hbm.at[pl.ds(gbase + ci * _GCH, _GCH)],
                         buf[0], buf[2])

    def wait_idx(buf):
        pltpu.make_async_copy(idx_hbm.at[pl.ds(0, _GCH)], buf[0],
                              buf[2]).wait()

    def start_gather(buf):
        pltpu.async_copy(mat_hbm.at[buf[0]], buf[1], buf[3])

    def wait_gather(buf):
        pltpu.make_async_copy(mat_hbm.at[buf[0]], buf[1], buf[3]).wait()

    def start_write(ci, buf):
        pltpu.async_copy(buf[1], out_hbm.at[pl.ds(gbase + ci * _GCH, _GCH)],
                         buf[4])

    def wait_write(buf):
        pltpu.make_async_copy(buf[1], out_hbm.at[pl.ds(0, _GCH)],
                              buf[4]).wait()

    start_idx(0, bufs[0])
    wait_idx(bufs[0])
    start_gather(bufs[0])
    start_idx(1, bufs[1])

    def half(u, off):
        ci = 2 * u + off
        p = bufs[off]
        q = bufs[1 - off]
        wait_gather(p)
        start_write(ci, p)

        @pl.when(ci + 1 < _GNCH)
        def _():
            wait_idx(q)

            @pl.when(ci >= 1)
            def _():
                wait_write(q)

            start_gather(q)

        @pl.when(ci + 2 < _GNCH)
        def _():
            start_idx(ci + 2, p)

    def body2(u, carry):
        half(u, 0)
        half(u, 1)
        return carry

    lax.fori_loop(0, _GNCH // 2, body2, 0)
    wait_gather(bufs[(_GNCH - 1) % 2])
    start_write(_GNCH - 1, bufs[(_GNCH - 1) % 2])
    wait_write(bufs[0])
    wait_write(bufs[1])


def _loss_body(s_ref, p_ref, n_ref, epw_ref, pw_ref, pb_ref, w0_ref, b0_ref,
               w1_ref, b1_ref, ow_ref, ob_ref, cl_ref, reg_ref):
    i = pl.program_id(0)

    @pl.when(i == 0)
    def _init():
        cl_ref[...] = jnp.zeros((1, 1), jnp.float32)
        reg = jnp.zeros((), jnp.float32)
        for r in (epw_ref, pw_ref, pb_ref, w0_ref, b0_ref, w1_ref, b1_ref,
                  ow_ref, ob_ref):
            v = r[...]
            reg = reg + jnp.sum(v * v)
        reg_ref[...] = jnp.reshape(reg * _LAMBDA1, (1, 1))

    sm = s_ref[...]
    pm = p_ref[...]
    na = jnp.maximum(jnp.sqrt(jnp.sum(sm * sm, axis=1, keepdims=True)), 1e-8)
    nb = jnp.maximum(jnp.sqrt(jnp.sum(pm * pm, axis=1, keepdims=True)), 1e-8)
    ps = jnp.sum(sm * pm, axis=1, keepdims=True) / (na * nb)
    a = jnp.exp(ps / _TEMP)
    total = jnp.zeros((), jnp.float32)
    for k in range(_K):
        nk = n_ref[k]
        nn = jnp.maximum(jnp.sqrt(jnp.sum(nk * nk, axis=1, keepdims=True)),
                         1e-8)
        ns = jnp.sum(sm * nk, axis=1, keepdims=True) / (na * nn)
        bk = jnp.exp(ns / _TEMP)
        total = total + jnp.sum(-jnp.log(a / (a + bk + 1e-8)))
    cl_ref[...] = cl_ref[...] + jnp.reshape(total / (_B * _K), (1, 1))


def _loss(rows_s, rows_p, rows_n3, emb_p_w, proj_W, pb2, W0, b02, W1, b12,
          out_W, ob2):
    grid = (_B // _BB,)
    cl, reg = pl.pallas_call(
        _loss_body,
        grid=grid,
        in_specs=[
            pl.BlockSpec((_BB, _D), lambda i: (i, 0)),
            pl.BlockSpec((_BB, _D), lambda i: (i, 0)),
            pl.BlockSpec((_K, _BB, _D), lambda i: (0, i, 0)),
            pl.BlockSpec((_DEPTH, _PD), lambda i: (0, 0)),
            pl.BlockSpec((_D, _D + _PD), lambda i: (0, 0)),
            pl.BlockSpec((1, _D), lambda i: (0, 0)),
            pl.BlockSpec((_D, _D), lambda i: (0, 0)),
            pl.BlockSpec((1, _D), lambda i: (0, 0)),
            pl.BlockSpec((_D, _D), lambda i: (0, 0)),
            pl.BlockSpec((1, _D), lambda i: (0, 0)),
            pl.BlockSpec((_D, _D), lambda i: (0, 0)),
            pl.BlockSpec((1, _D), lambda i: (0, 0)),
        ],
        out_specs=[
            pl.BlockSpec((1, 1), lambda i: (0, 0)),
            pl.BlockSpec((1, 1), lambda i: (0, 0)),
        ],
        out_shape=[
            jax.ShapeDtypeStruct((1, 1), jnp.float32),
            jax.ShapeDtypeStruct((1, 1), jnp.float32),
        ],
    )(rows_s, rows_p, rows_n3, emb_p_w, proj_W, pb2, W0, b02, W1, b12,
      out_W, ob2)
    return cl, reg


def kernel(emb_s, edge_index, adj_values, position_ids, sids, pos, negs,
           emb_p_w, proj_W, proj_b, W0, b0, W1, b1, out_W, out_b):
    f32 = jnp.float32
    i32 = jnp.int32
    emb_s = emb_s.astype(f32)
    idx_i = edge_index[0].astype(i32)
    idx_j = edge_index[1].astype(i32)
    vals = adj_values.astype(f32)

    pad_e = _EPAD - _E
    idx_i_p = jnp.pad(idx_i, (0, pad_e))
    idx_j_p = jnp.pad(idx_j, (0, pad_e))
    vals_p = jnp.pad(vals, (0, pad_e))
    nblk = _EPAD // _CH
    meta = jnp.stack(
        [idx_j_p.reshape(nblk, _CH), idx_i_p.reshape(nblk, _CH),
         lax.bitcast_convert_type(vals_p, i32).reshape(nblk, _CH)],
        axis=1).reshape(-1)

    onehot = (position_ids.astype(i32)[:, None]
              == jnp.arange(_DEPTH, dtype=i32)[None, :]).astype(f32)
    zeros_blk = jnp.zeros((_ZROWS, _D), f32)

    pb2 = proj_b.reshape(1, _D).astype(f32)
    b02 = b0.reshape(1, _D).astype(f32)
    b12 = b1.reshape(1, _D).astype(f32)
    ob2 = out_b.reshape(1, _D).astype(f32)

    x0, h0 = _proj0(emb_s, onehot, emb_p_w.astype(f32), proj_W.astype(f32),
                    pb2, W0.astype(f32), b02)
    p = _sc_spmm(h0, meta, zeros_blk)
    x1, h1 = _residual_layer(x0, p[0, :_N], p[1, :_N], W1.astype(f32), b12,
                             want_xn=True)
    q = _sc_spmm(h1, meta, zeros_blk)
    out = _residual_layer(x1, q[0, :_N], q[1, :_N], out_W.astype(f32), ob2,
                          want_xn=False)

    gidx = jnp.concatenate([sids.astype(i32), pos.astype(i32),
                            negs.astype(i32).reshape(-1)])
    rows = _sc_gather(out, gidx)
    rows_s = rows[:_B]
    rows_p = rows[_B:2 * _B]
    rows_n3 = rows[2 * _B:].reshape(_K, _B, _D)

    cl, reg = _loss(rows_s, rows_p, rows_n3, emb_p_w.astype(f32),
                    proj_W.astype(f32), pb2, W0.astype(f32), b02,
                    W1.astype(f32), b12, out_W.astype(f32), ob2)
    loss_cl = cl[0, 0]
    loss_reg = reg[0, 0]
    return (loss_cl + loss_reg, loss_cl, loss_reg)


# speedup vs baseline: 3.3964x; 1.0629x over previous
import functools

import jax
import jax.numpy as jnp
from jax import lax
from jax.experimental import pallas as pl
from jax.experimental.pallas import tpu as pltpu
from jax.experimental.pallas import tpu_sc as plsc

_N = 10000
_E = 320000
_D = 128
_PD = 64
_DEPTH = 16
_B = 1024
_K = 32
_TEMP = 0.5
_LAMBDA1 = 1e-05

_NPAD = 10112
_EPT = 10240
_EPAD = 32 * _EPT
_CH = 64
_NCHUNK = _EPT // _CH
_ZROWS = _NPAD // 16
_G = _B * (_K + 2)
_GPT = _G // 32
_GCH = 64
_GNCH = _GPT // _GCH

_BN = 2000
_BB = 256


def _mm_nt(a, b):
    return lax.dot_general(a, b, (((1,), (1,)), ((), ())),
                           preferred_element_type=jnp.float32)


def _mm_nn(a, b):
    return lax.dot_general(a, b, (((1,), (0,)), ((), ())),
                           preferred_element_type=jnp.float32)


def _proj0_body(es_ref, oh_ref, epw_ref, pw_ref, pb_ref, w0_ref, b0_ref,
                x0_ref, h0_ref):
    pw = pw_ref[...]
    wa = pw[:, :_D]
    wb = pw[:, _D:]
    emb_pp = _mm_nt(epw_ref[...], wb)
    x0 = _mm_nt(es_ref[...], wa) + _mm_nn(oh_ref[...], emb_pp) + pb_ref[...]
    h0 = jnp.maximum(_mm_nt(x0, w0_ref[...]) + b0_ref[...], 0.0)
    x0_ref[...] = x0
    h0_ref[...] = h0


def _proj0(emb_s, onehot, emb_p_w, proj_W, pb2, W0, b02):
    grid = (_N // _BN,)
    return pl.pallas_call(
        _proj0_body,
        grid=grid,
        in_specs=[
            pl.BlockSpec((_BN, _D), lambda i: (i, 0)),
            pl.BlockSpec((_BN, _DEPTH), lambda i: (i, 0)),
            pl.BlockSpec((_DEPTH, _PD), lambda i: (0, 0)),
            pl.BlockSpec((_D, _D + _PD), lambda i: (0, 0)),
            pl.BlockSpec((1, _D), lambda i: (0, 0)),
            pl.BlockSpec((_D, _D), lambda i: (0, 0)),
            pl.BlockSpec((1, _D), lambda i: (0, 0)),
        ],
        out_specs=[
            pl.BlockSpec((_BN, _D), lambda i: (i, 0)),
            pl.BlockSpec((_BN, _D), lambda i: (i, 0)),
        ],
        out_shape=[
            jax.ShapeDtypeStruct((_N, _D), jnp.float32),
            jax.ShapeDtypeStruct((_N, _D), jnp.float32),
        ],
    )(emb_s, onehot, emb_p_w, proj_W, pb2, W0, b02)


def _res_body_both(x_ref, p0_ref, p1_ref, w_ref, b_ref, xn_ref, y_ref):
    xn = x_ref[...] + p0_ref[0] + p1_ref[0]
    y = jnp.maximum(_mm_nt(xn, w_ref[...]) + b_ref[...], 0.0)
    xn_ref[...] = xn
    y_ref[...] = y


def _res_body_out(x_ref, p0_ref, p1_ref, w_ref, b_ref, y_ref):
    xn = x_ref[...] + p0_ref[0] + p1_ref[0]
    y_ref[...] = _mm_nt(xn, w_ref[...]) + b_ref[...]


def _residual_layer(x, p, W, b2, want_xn):
    grid = (_N // _BN,)
    in_specs = [
        pl.BlockSpec((_BN, _D), lambda i: (i, 0)),
        pl.BlockSpec((1, _BN, _D), lambda i: (0, i, 0)),
        pl.BlockSpec((1, _BN, _D), lambda i: (1, i, 0)),
        pl.BlockSpec((_D, _D), lambda i: (0, 0)),
        pl.BlockSpec((1, _D), lambda i: (0, 0)),
    ]
    if want_xn:
        return pl.pallas_call(
            _res_body_both,
            grid=grid,
            in_specs=in_specs,
            out_specs=[pl.BlockSpec((_BN, _D), lambda i: (i, 0))] * 2,
            out_shape=[jax.ShapeDtypeStruct((_N, _D), jnp.float32)] * 2,
        )(x, p, p, W, b2)
    return pl.pallas_call(
        _res_body_out,
        grid=grid,
        in_specs=in_specs,
        out_specs=pl.BlockSpec((_BN, _D), lambda i: (i, 0)),
        out_shape=jax.ShapeDtypeStruct((_N, _D), jnp.float32),
    )(x, p, p, W, b2)


_SC_MESH = plsc.VectorSubcoreMesh(core_axis_name="c", subcore_axis_name="s")

_MW = 3 * _CH


@functools.partial(
    pl.kernel,
    out_type=jax.ShapeDtypeStruct((2, _NPAD, _D), jnp.float32),
    mesh=_SC_MESH,
    scratch_types=(
        [pltpu.VMEM_SHARED((_NPAD, _D), jnp.float32)]
        + [pltpu.VMEM((_MW,), jnp.int32)] * 4
        + [pltpu.VMEM((_CH, _D), jnp.float32)] * 4
        + [pltpu.VMEM((_CH,), jnp.int32)] * 4
        + [pltpu.SemaphoreType.DMA] * 12
    ),
)
def _sc_spmm(h_hbm, meta_hbm, zeros_hbm, out_hbm,
             acc, m0, m1, m2, m3, r0, r1, r2, r3, i0, i1, i2, i3,
             sm0, sm1, sm2, sm3, sg0, sg1, sg2, sg3, ss0, ss1, ss2, ss3):
    c = lax.axis_index("c")
    s = lax.axis_index("s")
    wid = c * 16 + s
    pltpu.sync_copy(zeros_hbm, acc.at[pl.ds(s * _ZROWS, _ZROWS)])
    plsc.subcore_barrier()

    base = wid * _NCHUNK
    bufs = (
        (m0, r0, i0, sm0, sg0, ss0),
        (m1, r1, i1, sm1, sg1, ss1),
        (m2, r2, i2, sm2, sg2, ss2),
        (m3, r3, i3, sm3, sg3, ss3),
    )

    def start_meta(blk, buf):
        pltpu.async_copy(meta_hbm.at[pl.ds(blk * _MW, _MW)], buf[0], buf[3])

    def wait_meta(buf):
        pltpu.make_async_copy(meta_hbm.at[pl.ds(0, _MW)], buf[0],
                              buf[3]).wait()

    def start_gather(buf):
        pltpu.async_copy(h_hbm.at[buf[0].at[pl.ds(0, _CH)]], buf[1], buf[4])

    def wait_gather(buf):
        pltpu.make_async_copy(h_hbm.at[buf[0].at[pl.ds(0, _CH)]], buf[1],
                              buf[4]).wait()

    def start_scatter(buf):
        pltpu.async_copy(buf[1], acc.at[buf[2]], buf[5], add=True)

    def wait_scatter(buf):
        pltpu.make_async_copy(buf[1], acc.at[buf[2]], buf[5]).wait()

    def scale(buf):
        mv, rv = buf[0], buf[1]

        def grp(t, carry2):
            vseg = mv[pl.ds(2 * _CH + t * 16, 16)]
            for l in range(16):
                v = lax.bitcast_convert_type(vseg[l], jnp.float32)
                r = t * 16 + l
                for g in range(_D // 16):
                    sl = pl.ds(g * 16, 16)
                    rv[r, sl] = rv[r, sl] * v
            return carry2

        lax.fori_loop(0, _CH // 16, grp, 0)

    def copy_idxi(buf):
        mv, iv = buf[0], buf[2]
        for t in range(_CH // 16):
            iv[pl.ds(t * 16, 16)] = mv[pl.ds(_CH + t * 16, 16)]

    for b in range(4):
        start_meta(base + b, bufs[b])
    for b in range(2):
        wait_meta(bufs[b])
        start_gather(bufs[b])

    def quarter(u, off):
        ci = 4 * u + off
        p = bufs[off]
        nxt = bufs[(off + 2) % 4]
        wait_gather(p)

        @pl.when(ci + 2 < _NCHUNK)
        def _():
            wait_meta(nxt)

            @pl.when(ci >= 2)
            def _():
                wait_scatter(nxt)

            start_gather(nxt)

        scale(p)
        copy_idxi(p)
        start_scatter(p)

        @pl.when(ci + 4 < _NCHUNK)
        def _():
            start_meta(base + ci + 4, p)

    def body4(u, carry):
        for off in range(4):
            quarter(u, off)
        return carry

    lax.fori_loop(0, _NCHUNK // 4, body4, 0)
    for b in range(4):
        wait_scatter(bufs[b])
    plsc.subcore_barrier()
    pltpu.sync_copy(acc.at[pl.ds(s * _ZROWS, _ZROWS)],
                    out_hbm.at[c, pl.ds(s * _ZROWS, _ZROWS)])


@functools.partial(
    pl.kernel,
    out_type=jax.ShapeDtypeStruct((_G, _D), jnp.float32),
    mesh=_SC_MESH,
    scratch_types=(
        [pltpu.VMEM((_GCH,), jnp.int32)] * 2
        + [pltpu.VMEM((_GCH, _D), jnp.float32)] * 2
        + [pltpu.SemaphoreType.DMA] * 6
    ),
)
def _sc_gather(mat_hbm, idx_hbm, out_hbm, ia, ib, ra, rb,
               sia, sib, sga, sgb, swa, swb):
    c = lax.axis_index("c")
    s = lax.axis_index("s")
    wid = c * 16 + s
    gbase = wid * _GPT
    bufs = ((ia, ra, sia, sga, swa), (ib, rb, sib, sgb, swb))

    def start_idx(ci, buf):
        pltpu.async_copy(idx_hbm.at[pl.ds(gbase + ci * _GCH, _GCH)],
                         buf[0], buf[2])

    def wait_idx(buf):
        pltpu.make_async_copy(idx_hbm.at[pl.ds(0, _GCH)], buf[0],
                              buf[2]).wait()

    def start_gather(buf):
        pltpu.async_copy(mat_hbm.at[buf[0]], buf[1], buf[3])

    def wait_gather(buf):
        pltpu.make_async_copy(mat_hbm.at[buf[0]], buf[1], buf[3]).wait()

    def start_write(ci, buf):
        pltpu.async_copy(buf[1], out_hbm.at[pl.ds(gbase + ci * _GCH, _GCH)],
                         buf[4])

    def wait_write(buf):
        pltpu.make_async_copy(buf[1], out_hbm.at[pl.ds(0, _GCH)],
                              buf[4]).wait()

    start_idx(0, bufs[0])
    wait_idx(bufs[0])
    start_gather(bufs[0])
    start_idx(1, bufs[1])

    def half(u, off):
        ci = 2 * u + off
        p = bufs[off]
        q = bufs[1 - off]
        wait_gather(p)
        start_write(ci, p)

        @pl.when(ci + 1 < _GNCH)
        def _():
            wait_idx(q)

            @pl.when(ci >= 1)
            def _():
                wait_write(q)

            start_gather(q)

        @pl.when(ci + 2 < _GNCH)
        def _():
            start_idx(ci + 2, p)

    def body2(u, carry):
        half(u, 0)
        half(u, 1)
        return carry

    lax.fori_loop(0, _GNCH // 2, body2, 0)
    wait_gather(bufs[(_GNCH - 1) % 2])
    start_write(_GNCH - 1, bufs[(_GNCH - 1) % 2])
    wait_write(bufs[0])
    wait_write(bufs[1])


def _loss_body(s_ref, p_ref, n_ref, epw_ref, pw_ref, pb_ref, w0_ref, b0_ref,
               w1_ref, b1_ref, ow_ref, ob_ref, cl_ref, reg_ref):
    i = pl.program_id(0)

    @pl.when(i == 0)
    def _init():
        cl_ref[...] = jnp.zeros((1, 1), jnp.float32)
        reg = jnp.zeros((), jnp.float32)
        for r in (epw_ref, pw_ref, pb_ref, w0_ref, b0_ref, w1_ref, b1_ref,
                  ow_ref, ob_ref):
            v = r[...]
            reg = reg + jnp.sum(v * v)
        reg_ref[...] = jnp.reshape(reg * _LAMBDA1, (1, 1))

    sm = s_ref[...]
    pm = p_ref[...]
    na = jnp.maximum(jnp.sqrt(jnp.sum(sm * sm, axis=1, keepdims=True)), 1e-8)
    nb = jnp.maximum(jnp.sqrt(jnp.sum(pm * pm, axis=1, keepdims=True)), 1e-8)
    ps = jnp.sum(sm * pm, axis=1, keepdims=True) / (na * nb)
    a = jnp.exp(ps / _TEMP)
    total = jnp.zeros((), jnp.float32)
    for k in range(_K):
        nk = n_ref[k]
        nn = jnp.maximum(jnp.sqrt(jnp.sum(nk * nk, axis=1, keepdims=True)),
                         1e-8)
        ns = jnp.sum(sm * nk, axis=1, keepdims=True) / (na * nn)
        bk = jnp.exp(ns / _TEMP)
        total = total + jnp.sum(-jnp.log(a / (a + bk + 1e-8)))
    cl_ref[...] = cl_ref[...] + jnp.reshape(total / (_B * _K), (1, 1))


def _loss(rows_s, rows_p, rows_n3, emb_p_w, proj_W, pb2, W0, b02, W1, b12,
          out_W, ob2):
    grid = (_B // _BB,)
    cl, reg = pl.pallas_call(
        _loss_body,
        grid=grid,
        in_specs=[
            pl.BlockSpec((_BB, _D), lambda i: (i, 0)),
            pl.BlockSpec((_BB, _D), lambda i: (i, 0)),
            pl.BlockSpec((_K, _BB, _D), lambda i: (0, i, 0)),
            pl.BlockSpec((_DEPTH, _PD), lambda i: (0, 0)),
            pl.BlockSpec((_D, _D + _PD), lambda i: (0, 0)),
            pl.BlockSpec((1, _D), lambda i: (0, 0)),
            pl.BlockSpec((_D, _D), lambda i: (0, 0)),
            pl.BlockSpec((1, _D), lambda i: (0, 0)),
            pl.BlockSpec((_D, _D), lambda i: (0, 0)),
            pl.BlockSpec((1, _D), lambda i: (0, 0)),
            pl.BlockSpec((_D, _D), lambda i: (0, 0)),
            pl.BlockSpec((1, _D), lambda i: (0, 0)),
        ],
        out_specs=[
            pl.BlockSpec((1, 1), lambda i: (0, 0)),
            pl.BlockSpec((1, 1), lambda i: (0, 0)),
        ],
        out_shape=[
            jax.ShapeDtypeStruct((1, 1), jnp.float32),
            jax.ShapeDtypeStruct((1, 1), jnp.float32),
        ],
    )(rows_s, rows_p, rows_n3, emb_p_w, proj_W, pb2, W0, b02, W1, b12,
      out_W, ob2)
    return cl, reg


def kernel(emb_s, edge_index, adj_values, position_ids, sids, pos, negs,
           emb_p_w, proj_W, proj_b, W0, b0, W1, b1, out_W, out_b):
    f32 = jnp.float32
    i32 = jnp.int32
    emb_s = emb_s.astype(f32)
    idx_i = edge_index[0].astype(i32)
    idx_j = edge_index[1].astype(i32)
    vals = adj_values.astype(f32)

    pad_e = _EPAD - _E
    idx_i_p = jnp.pad(idx_i, (0, pad_e))
    idx_j_p = jnp.pad(idx_j, (0, pad_e))
    vals_p = jnp.pad(vals, (0, pad_e))
    nblk = _EPAD // _CH
    meta = jnp.stack(
        [idx_j_p.reshape(nblk, _CH), idx_i_p.reshape(nblk, _CH),
         lax.bitcast_convert_type(vals_p, i32).reshape(nblk, _CH)],
        axis=1).reshape(-1)

    onehot = (position_ids.astype(i32)[:, None]
              == jnp.arange(_DEPTH, dtype=i32)[None, :]).astype(f32)
    zeros_blk = jnp.zeros((_ZROWS, _D), f32)

    pb2 = proj_b.reshape(1, _D).astype(f32)
    b02 = b0.reshape(1, _D).astype(f32)
    b12 = b1.reshape(1, _D).astype(f32)
    ob2 = out_b.reshape(1, _D).astype(f32)

    x0, h0 = _proj0(emb_s, onehot, emb_p_w.astype(f32), proj_W.astype(f32),
                    pb2, W0.astype(f32), b02)
    p = _sc_spmm(h0, meta, zeros_blk)
    x1, h1 = _residual_layer(x0, p, W1.astype(f32), b12, want_xn=True)
    q = _sc_spmm(h1, meta, zeros_blk)
    out = _residual_layer(x1, q, out_W.astype(f32), ob2, want_xn=False)

    gidx = jnp.concatenate([sids.astype(i32), pos.astype(i32),
                            negs.astype(i32).reshape(-1)])
    rows = _sc_gather(out, gidx)
    rows_s = rows[:_B]
    rows_p = rows[_B:2 * _B]
    rows_n3 = rows[2 * _B:].reshape(_K, _B, _D)

    cl, reg = _loss(rows_s, rows_p, rows_n3, emb_p_w.astype(f32),
                    proj_W.astype(f32), pb2, W0.astype(f32), b02,
                    W1.astype(f32), b12, out_W.astype(f32), ob2)
    loss_cl = cl[0, 0]
    loss_reg = reg[0, 0]
    return (loss_cl + loss_reg, loss_cl, loss_reg)
